# speedup vs baseline: 1.4966x; 1.4966x over previous
import functools

import jax
import jax.numpy as jnp
from jax.experimental import pallas as pl

_BS = 512


def _add_block(x_ref, t_ref, o_ref):
    o_ref[...] = x_ref[...] + t_ref[...][None]


@functools.partial(jax.jit, static_argnums=(2,))
def _posemb_add(x, table_slice, seq_block):
    B, S, D = x.shape
    n_seq = S // seq_block
    return pl.pallas_call(
        _add_block,
        grid=(n_seq, B),
        in_specs=[
            pl.BlockSpec((1, seq_block, D), lambda i, b: (b, i, 0)),
            pl.BlockSpec((seq_block, D), lambda i, b: (i, 0)),
        ],
        out_specs=pl.BlockSpec((1, seq_block, D), lambda i, b: (b, i, 0)),
        out_shape=jax.ShapeDtypeStruct((B, S, D), x.dtype),
    )(x, table_slice)


def kernel(x, table, offset=0):
    S = x.shape[1]
    table_slice = jax.lax.dynamic_slice_in_dim(table, offset, S, axis=0)
    return _posemb_add(x, table_slice, _BS)


# speedup vs baseline: 1.6669x; 1.1138x over previous
import functools

import jax
import jax.numpy as jnp
from jax.experimental import pallas as pl

_BS = 1024


def _add_block(x_ref, t_ref, o_ref):
    o_ref[...] = x_ref[...] + t_ref[...][None]


@functools.partial(jax.jit, static_argnums=(2,))
def _posemb_add(x, table_slice, seq_block):
    B, S, D = x.shape
    n_seq = S // seq_block
    return pl.pallas_call(
        _add_block,
        grid=(n_seq, B),
        in_specs=[
            pl.BlockSpec((1, seq_block, D), lambda i, b: (b, i, 0)),
            pl.BlockSpec((seq_block, D), lambda i, b: (i, 0)),
        ],
        out_specs=pl.BlockSpec((1, seq_block, D), lambda i, b: (b, i, 0)),
        out_shape=jax.ShapeDtypeStruct((B, S, D), x.dtype),
    )(x, table_slice)


def kernel(x, table, offset=0):
    S = x.shape[1]
    table_slice = jax.lax.dynamic_slice_in_dim(table, offset, S, axis=0)
    return _posemb_add(x, table_slice, _BS)


# speedup vs baseline: 1.7326x; 1.0394x over previous
import functools

import jax
import jax.numpy as jnp
from jax.experimental import pallas as pl

_BS = 2048


def _add_block(x_ref, t_ref, o_ref):
    o_ref[...] = x_ref[...] + t_ref[...][None]


@functools.partial(jax.jit, static_argnums=(2,))
def _posemb_add(x, table_slice, seq_block):
    B, S, D = x.shape
    n_seq = S // seq_block
    return pl.pallas_call(
        _add_block,
        grid=(n_seq, B),
        in_specs=[
            pl.BlockSpec((1, seq_block, D), lambda i, b: (b, i, 0)),
            pl.BlockSpec((seq_block, D), lambda i, b: (i, 0)),
        ],
        out_specs=pl.BlockSpec((1, seq_block, D), lambda i, b: (b, i, 0)),
        out_shape=jax.ShapeDtypeStruct((B, S, D), x.dtype),
    )(x, table_slice)


def kernel(x, table, offset=0):
    S = x.shape[1]
    table_slice = jax.lax.dynamic_slice_in_dim(table, offset, S, axis=0)
    return _posemb_add(x, table_slice, _BS)


# speedup vs baseline: 1.7399x; 1.0042x over previous
import functools

import jax
import jax.numpy as jnp
from jax.experimental import pallas as pl
from jax.experimental.pallas import tpu as pltpu

_BS = 2048


def _add_block(x_ref, t_ref, o_ref):
    o_ref[...] = x_ref[...] + t_ref[...][None]


@functools.partial(jax.jit, static_argnums=(2,))
def _posemb_add(x, table_slice, seq_block):
    B, S, D = x.shape
    n_seq = S // seq_block
    return pl.pallas_call(
        _add_block,
        grid=(n_seq, B),
        in_specs=[
            pl.BlockSpec((1, seq_block, D), lambda i, b: (b, i, 0)),
            pl.BlockSpec((seq_block, D), lambda i, b: (i, 0)),
        ],
        out_specs=pl.BlockSpec((1, seq_block, D), lambda i, b: (b, i, 0)),
        out_shape=jax.ShapeDtypeStruct((B, S, D), x.dtype),
        compiler_params=pltpu.CompilerParams(
            dimension_semantics=("parallel", "parallel"),
            vmem_limit_bytes=128 * 1024 * 1024,
        ),
    )(x, table_slice)


def kernel(x, table, offset=0):
    S = x.shape[1]
    table_slice = jax.lax.dynamic_slice_in_dim(table, offset, S, axis=0)
    return _posemb_add(x, table_slice, _BS)
